# baseline (speedup 1.0000x reference)
import functools

import jax
import jax.numpy as jnp
from jax import lax
from jax.experimental import pallas as pl
from jax.experimental.pallas import tpu as pltpu
from jax.experimental.pallas import tpu_sc as plsc

_K = 8
_NEG_INF = float("-inf")
_NW = 32
_SEGS = 4
_GL = 128


def _normalize_body(x_ref, o_ref):
    x = x_ref[...]
    n = jnp.sqrt(jnp.sum(x * x, axis=1, keepdims=True))
    o_ref[...] = x / jnp.maximum(n, 1e-12)


_SORT8 = [
    (0, 1), (2, 3), (4, 5), (6, 7),
    (0, 2), (1, 3), (4, 6), (5, 7),
    (1, 2), (5, 6), (0, 4), (1, 5),
    (2, 6), (3, 7), (2, 4), (3, 5),
    (1, 2), (3, 4), (5, 6),
]


def _ce(vals, chs, a, b):
    va, vb = vals[a], vals[b]
    g = va < vb
    vals[a], vals[b] = jnp.maximum(va, vb), jnp.minimum(va, vb)
    ca, cb = chs[a], chs[b]
    chs[a], chs[b] = jnp.where(g, cb, ca), jnp.where(g, ca, cb)


def _simtopk_body(a_ref, b_ref, idx_ref, *, strip: int, n: int):
    a = a_ref[...]
    b = b_ref[...]
    sim = jax.lax.dot_general(
        a, b, (((1,), (1,)), ((), ())), preferred_element_type=jnp.float32)
    i = pl.program_id(0)
    row_ids = jax.lax.broadcasted_iota(jnp.int32, (strip, n), 0) + i * strip
    col_ids = jax.lax.broadcasted_iota(jnp.int32, (strip, n), 1)
    sim = jnp.where(col_ids == row_ids, _NEG_INF, sim)

    ngrp = (n // 128) // _K
    tops = []
    for g in range(ngrp):
        vals = [sim[:, (g * _K + s) * 128:(g * _K + s + 1) * 128]
                for s in range(_K)]
        chs = [jnp.full((strip, 128), g * _K + s, jnp.int32)
               for s in range(_K)]
        for (x, y) in _SORT8:
            _ce(vals, chs, x, y)
        tops.append((vals, chs))
    while len(tops) > 1:
        nxt = []
        for p in range(0, len(tops), 2):
            (va, ca), (vb, cb) = tops[p], tops[p + 1]
            mv, mc = [], []
            for s in range(_K):
                x, y = va[s], vb[_K - 1 - s]
                gt = x < y
                mv.append(jnp.maximum(x, y))
                mc.append(jnp.where(gt, cb[_K - 1 - s], ca[s]))
            for stage in (4, 2, 1):
                for lo in range(_K):
                    if lo & stage == 0 and lo + stage < _K:
                        _ce(mv, mc, lo, lo + stage)
            nxt.append((mv, mc))
        tops = nxt
    vals, chs = tops[0]
    lane = jax.lax.broadcasted_iota(jnp.int32, (strip, 128), 1)
    cols = [chs[s] * 128 + lane for s in range(_K)]

    out_cols = []
    for _ in range(_K):
        m = vals[0]
        for s in range(1, _K):
            m = jnp.maximum(m, vals[s])
        m = jnp.max(m, axis=1, keepdims=True)
        amin = jnp.full((strip, 128), n, jnp.int32)
        for s in range(_K):
            amin = jnp.minimum(amin, jnp.where(vals[s] == m, cols[s], n))
        amax = jnp.min(amin, axis=1, keepdims=True)
        out_cols.append(amax)
        for s in range(_K):
            vals[s] = jnp.where(cols[s] == amax, _NEG_INF, vals[s])
    idx_ref[...] = jnp.concatenate(out_cols, axis=1)


def _sc_adj_body(idx_hbm, out_hbm, idx_v, rows2d, vals2d, cnt_v, val_v,
                 zsrc, sem_fill, sem_sc, *, n: int):
    rpw = n // _NW
    nedge = n * _K
    seg_elems = nedge // _SEGS
    gps = seg_elems // _GL
    slab = rpw * n
    zlen = min(16384, slab)
    nfill = slab // zlen
    fwd_groups = (rpw * _K) // _GL

    c = lax.axis_index("c")
    s = lax.axis_index("s")
    w = s * 2 + c
    base = w * rpw
    iota = lax.iota(jnp.int32, 16)

    def zero_body(q, _):
        zsrc[pl.ds(q * 16, 16)] = jnp.zeros((16,), jnp.float32)
        return _
    lax.fori_loop(0, zlen // 16, zero_body, None)

    def fill_issue(i, _):
        pltpu.async_copy(zsrc, out_hbm.at[pl.ds(base * n + i * zlen, zlen)],
                         sem_fill)
        return _
    lax.fori_loop(0, nfill, fill_issue, None)

    pltpu.sync_copy(idx_hbm, idx_v)

    def cnt_init(q, _):
        cnt_v[pl.ds(q * 16, 16)] = jnp.full((16,), float(_K), jnp.float32)
        return _
    lax.fori_loop(0, rpw // 16, cnt_init, None)

    def s1_body(e16, _):
        e0 = e16 * 16
        t = idx_v[pl.ds(e0, 16)]
        inr = (t >= base) & (t < base + rpw)
        nhit = jnp.sum(jnp.where(inr, 1, 0))

        @pl.when(nhit > 0)
        def _do():
            src = lax.shift_right_logical(e0 + iota, 3)
            t8 = t * _K
            mut = plsc.load_gather(idx_v, [t8]) == src
            for k in range(1, _K):
                mut = mut | (plsc.load_gather(idx_v, [t8 + k]) == src)
            novel = jnp.where(inr & (~mut), 1.0, 0.0).astype(jnp.float32)
            tl = jnp.where(inr, t - base, 0)
            for l in range(16):
                plsc.addupdate_scatter(cnt_v, [tl], novel,
                                       mask=(iota == l) & inr)
        return _
    lax.fori_loop(0, nedge // 16, s1_body, None)

    def val_body(q, _):
        val_v[pl.ds(q * 16, 16)] = 1.0 / cnt_v[pl.ds(q * 16, 16)]
        return _
    lax.fori_loop(0, rpw // 16, val_body, None)

    def fill_drain(i, _):
        pltpu.make_async_copy(
            zsrc, out_hbm.at[pl.ds(base * n + i * zlen, zlen)],
            sem_fill).wait()
        return _
    lax.fori_loop(0, nfill, fill_drain, None)

    def fw_body(g, _):
        for q in range(_GL // 16):
            el = q * 16 + g * _GL
            t = idx_v[pl.ds(base * _K + el, 16)]
            i_loc = lax.shift_right_logical(el + iota, 3)
            rows2d[g, pl.ds(q * 16, 16)] = (base + i_loc) * n + t
            vals2d[g, pl.ds(q * 16, 16)] = plsc.load_gather(val_v, [i_loc])
        return _
    lax.fori_loop(0, fwd_groups, fw_body, None)

    def fw_issue(g, _):
        pltpu.async_copy(vals2d.at[g], out_hbm.at[rows2d.at[g]], sem_sc)
        return _
    lax.fori_loop(0, fwd_groups, fw_issue, None)

    def fw_drain(g, _):
        pltpu.make_async_copy(vals2d.at[g], out_hbm.at[rows2d.at[g]],
                              sem_sc).wait()
        return _
    lax.fori_loop(0, fwd_groups, fw_drain, None)

    for seg in range(_SEGS):
        def s2_body(g, _):
            for q in range(_GL // 16):
                e0 = seg * seg_elems + q * 16
                t = idx_v[pl.ds(e0 + g * _GL, 16)]
                src = lax.shift_right_logical(e0 + g * _GL + iota, 3)
                inr = (t >= base) & (t < base + rpw)
                tl = jnp.where(inr, t - base, 0)
                diag = (base + q * 16 + iota) * (n + 1)
                off = jnp.where(inr, t * n + src, diag)
                v = jnp.where(inr, plsc.load_gather(val_v, [tl]), 0.0)
                rows2d[g, pl.ds(q * 16, 16)] = off
                vals2d[g, pl.ds(q * 16, 16)] = v.astype(jnp.float32)
            return _
        lax.fori_loop(0, gps, s2_body, None)

        def s2_issue(g, _):
            pltpu.async_copy(vals2d.at[g], out_hbm.at[rows2d.at[g]], sem_sc)
            return _
        lax.fori_loop(0, gps, s2_issue, None)

        def s2_drain(g, _):
            pltpu.make_async_copy(vals2d.at[g], out_hbm.at[rows2d.at[g]],
                                  sem_sc).wait()
            return _
        lax.fori_loop(0, gps, s2_drain, None)


def _sc_adjacency(idx_flat, n):
    mesh = plsc.VectorSubcoreMesh(
        core_axis_name="c", subcore_axis_name="s", num_cores=2,
        num_subcores=16)
    kern = functools.partial(
        pl.kernel,
        out_type=jax.ShapeDtypeStruct((n * n,), jnp.float32),
        mesh=mesh,
        compiler_params=pltpu.CompilerParams(needs_layout_passes=False),
        scratch_types=[
            pltpu.VMEM((n * _K,), jnp.int32),
            pltpu.VMEM((_GL, _GL), jnp.int32),
            pltpu.VMEM((_GL, _GL), jnp.float32),
            pltpu.VMEM((n // _NW,), jnp.float32),
            pltpu.VMEM((n // _NW,), jnp.float32),
            pltpu.VMEM((min(16384, n * n // _NW),), jnp.float32),
            pltpu.SemaphoreType.DMA,
            pltpu.SemaphoreType.DMA,
        ],
    )(functools.partial(_sc_adj_body, n=n))
    return kern(idx_flat)


@jax.jit
def kernel(embeddings):
    n, d = embeddings.shape
    rn = min(1024, n)
    xn = pl.pallas_call(
        _normalize_body,
        grid=(n // rn,),
        in_specs=[pl.BlockSpec((rn, d), lambda i: (i, 0))],
        out_specs=pl.BlockSpec((rn, d), lambda i: (i, 0)),
        out_shape=jax.ShapeDtypeStruct((n, d), jnp.float32),
    )(embeddings)

    strip = min(256, n)
    nstrips = n // strip
    idx = pl.pallas_call(
        functools.partial(_simtopk_body, strip=strip, n=n),
        grid=(nstrips,),
        in_specs=[
            pl.BlockSpec((strip, d), lambda i: (i, 0)),
            pl.BlockSpec((n, d), lambda i: (0, 0)),
        ],
        out_specs=pl.BlockSpec((strip, _K), lambda i: (i, 0)),
        out_shape=jax.ShapeDtypeStruct((n, _K), jnp.int32),
    )(xn, xn)

    adjacency = _sc_adjacency(idx.reshape(-1), n).reshape(n, n)
    return adjacency, xn


# speedup vs baseline: 3.6587x; 3.6587x over previous
import functools

import jax
import jax.numpy as jnp
from jax import lax
from jax.experimental import pallas as pl
from jax.experimental.pallas import tpu as pltpu
from jax.experimental.pallas import tpu_sc as plsc

_K = 8
_NEG_INF = float("-inf")
_NW = 32
_SEGS = 4
_GL = 128


def _normalize_body(x_ref, o_ref):
    x = x_ref[...]
    n = jnp.sqrt(jnp.sum(x * x, axis=1, keepdims=True))
    o_ref[...] = x / jnp.maximum(n, 1e-12)


_SORT8 = [
    (0, 1), (2, 3), (4, 5), (6, 7),
    (0, 2), (1, 3), (4, 6), (5, 7),
    (1, 2), (5, 6), (0, 4), (1, 5),
    (2, 6), (3, 7), (2, 4), (3, 5),
    (1, 2), (3, 4), (5, 6),
]


def _ce(vals, chs, a, b):
    va, vb = vals[a], vals[b]
    g = va < vb
    vals[a], vals[b] = jnp.maximum(va, vb), jnp.minimum(va, vb)
    ca, cb = chs[a], chs[b]
    chs[a], chs[b] = jnp.where(g, cb, ca), jnp.where(g, ca, cb)


def _simtopk_body(a_ref, b_ref, idx_ref, *, strip: int, n: int):
    a = a_ref[...]
    b = b_ref[...]
    sim = jax.lax.dot_general(
        a, b, (((1,), (1,)), ((), ())), preferred_element_type=jnp.float32)
    i = pl.program_id(0)
    row_ids = jax.lax.broadcasted_iota(jnp.int32, (strip, n), 0) + i * strip
    col_ids = jax.lax.broadcasted_iota(jnp.int32, (strip, n), 1)
    sim = jnp.where(col_ids == row_ids, _NEG_INF, sim)

    ngrp = (n // 128) // _K
    tops = []
    for g in range(ngrp):
        vals = [sim[:, (g * _K + s) * 128:(g * _K + s + 1) * 128]
                for s in range(_K)]
        chs = [jnp.full((strip, 128), g * _K + s, jnp.int32)
               for s in range(_K)]
        for (x, y) in _SORT8:
            _ce(vals, chs, x, y)
        tops.append((vals, chs))
    while len(tops) > 1:
        nxt = []
        for p in range(0, len(tops), 2):
            (va, ca), (vb, cb) = tops[p], tops[p + 1]
            mv, mc = [], []
            for s in range(_K):
                x, y = va[s], vb[_K - 1 - s]
                gt = x < y
                mv.append(jnp.maximum(x, y))
                mc.append(jnp.where(gt, cb[_K - 1 - s], ca[s]))
            for stage in (4, 2, 1):
                for lo in range(_K):
                    if lo & stage == 0 and lo + stage < _K:
                        _ce(mv, mc, lo, lo + stage)
            nxt.append((mv, mc))
        tops = nxt
    vals, chs = tops[0]
    lane = jax.lax.broadcasted_iota(jnp.int32, (strip, 128), 1)
    cols = [chs[s] * 128 + lane for s in range(_K)]

    out_cols = []
    for _ in range(_K):
        m = vals[0]
        for s in range(1, _K):
            m = jnp.maximum(m, vals[s])
        m = jnp.max(m, axis=1, keepdims=True)
        amin = jnp.full((strip, 128), n, jnp.int32)
        for s in range(_K):
            amin = jnp.minimum(amin, jnp.where(vals[s] == m, cols[s], n))
        amax = jnp.min(amin, axis=1, keepdims=True)
        out_cols.append(amax)
        for s in range(_K):
            vals[s] = jnp.where(cols[s] == amax, _NEG_INF, vals[s])
    idx_ref[...] = jnp.concatenate(out_cols, axis=1)


def _adjacency_body(idx_strip_ref, idxt_ref, out_ref, *, strip: int, n: int):
    idx_strip = idx_strip_ref[...]
    idxt = idxt_ref[...]
    i0 = pl.program_id(0) * strip
    row_ids = jax.lax.broadcasted_iota(jnp.int32, (strip, n), 0) + i0
    col_ids = jax.lax.broadcasted_iota(jnp.int32, (strip, n), 1)
    acc = jnp.zeros((strip, n), dtype=jnp.bool_)
    for k in range(_K):
        fwd = col_ids == idx_strip[:, k:k + 1]
        bwd = row_ids == idxt[k:k + 1, :]
        acc = acc | fwd | bwd
    a = acc.astype(jnp.float32)
    rs = jnp.maximum(jnp.sum(a, axis=1, keepdims=True), 1e-8)
    out_ref[...] = a / rs


def _sc_adj_body(idx_hbm, out_hbm, idx_v, rows2d, vals2d, cnt_v, val_v,
                 zsrc, sem_fill, sem_sc, *, n: int):
    rpw = n // _NW
    nedge = n * _K
    seg_elems = nedge // _SEGS
    gps = seg_elems // _GL
    slab = rpw * n
    zlen = min(16384, slab)
    nfill = slab // zlen
    fwd_groups = (rpw * _K) // _GL

    c = lax.axis_index("c")
    s = lax.axis_index("s")
    w = s * 2 + c
    base = w * rpw
    iota = lax.iota(jnp.int32, 16)

    def zero_body(q, _):
        zsrc[pl.ds(q * 16, 16)] = jnp.zeros((16,), jnp.float32)
        return _
    lax.fori_loop(0, zlen // 16, zero_body, None)

    def fill_issue(i, _):
        pltpu.async_copy(zsrc, out_hbm.at[pl.ds(base * n + i * zlen, zlen)],
                         sem_fill)
        return _
    lax.fori_loop(0, nfill, fill_issue, None)

    pltpu.sync_copy(idx_hbm, idx_v)

    def cnt_init(q, _):
        cnt_v[pl.ds(q * 16, 16)] = jnp.full((16,), float(_K), jnp.float32)
        return _
    lax.fori_loop(0, rpw // 16, cnt_init, None)

    def s1_body(e16, _):
        e0 = e16 * 16
        t = idx_v[pl.ds(e0, 16)]
        inr = (t >= base) & (t < base + rpw)
        nhit = jnp.sum(jnp.where(inr, 1, 0))

        @pl.when(nhit > 0)
        def _do():
            src = lax.shift_right_logical(e0 + iota, 3)
            t8 = t * _K
            mut = plsc.load_gather(idx_v, [t8]) == src
            for k in range(1, _K):
                mut = mut | (plsc.load_gather(idx_v, [t8 + k]) == src)
            novel = jnp.where(inr & (~mut), 1.0, 0.0).astype(jnp.float32)
            tl = jnp.where(inr, t - base, 0)
            for l in range(16):
                plsc.addupdate_scatter(cnt_v, [tl], novel,
                                       mask=(iota == l) & inr)
        return _
    lax.fori_loop(0, nedge // 16, s1_body, None)

    def val_body(q, _):
        val_v[pl.ds(q * 16, 16)] = 1.0 / cnt_v[pl.ds(q * 16, 16)]
        return _
    lax.fori_loop(0, rpw // 16, val_body, None)

    def fill_drain(i, _):
        pltpu.make_async_copy(
            zsrc, out_hbm.at[pl.ds(base * n + i * zlen, zlen)],
            sem_fill).wait()
        return _
    lax.fori_loop(0, nfill, fill_drain, None)

    def fw_body(g, _):
        for q in range(_GL // 16):
            el = q * 16 + g * _GL
            t = idx_v[pl.ds(base * _K + el, 16)]
            i_loc = lax.shift_right_logical(el + iota, 3)
            rows2d[g, pl.ds(q * 16, 16)] = (base + i_loc) * n + t
            vals2d[g, pl.ds(q * 16, 16)] = plsc.load_gather(val_v, [i_loc])
        return _
    lax.fori_loop(0, fwd_groups, fw_body, None)

    def fw_issue(g, _):
        pltpu.async_copy(vals2d.at[g], out_hbm.at[rows2d.at[g]], sem_sc)
        return _
    lax.fori_loop(0, fwd_groups, fw_issue, None)

    def fw_drain(g, _):
        pltpu.make_async_copy(vals2d.at[g], out_hbm.at[rows2d.at[g]],
                              sem_sc).wait()
        return _
    lax.fori_loop(0, fwd_groups, fw_drain, None)

    diag_pad = (base + iota) * (n + 1)

    def s2_flush(slot):
        pltpu.async_copy(vals2d.at[slot], out_hbm.at[rows2d.at[slot]],
                         sem_sc)

    def s2_drainslot(slot):
        pltpu.make_async_copy(vals2d.at[slot], out_hbm.at[rows2d.at[slot]],
                              sem_sc).wait()

    def s2_body(e16, carry):
        lo, gout = carry
        e0 = e16 * 16
        t = idx_v[pl.ds(e0, 16)]
        src = lax.shift_right_logical(e0 + iota, 3)
        inr = (t >= base) & (t < base + rpw)
        nhit = jnp.sum(jnp.where(inr, 1, 0))
        slot = lax.rem(gout, 128)
        tl = jnp.where(inr, t - base, 0)
        off = t * n + src
        v = plsc.load_gather(val_v, [tl])
        plsc.store_compressed(rows2d.at[slot, pl.ds(lo, 16)], off, mask=inr)
        plsc.store_compressed(vals2d.at[slot, pl.ds(lo, 16)], v, mask=inr)
        lo = lo + nhit

        @pl.when(lo >= 112)
        def _flush():
            pad_mask = (iota + 112) >= lo
            plsc.store_scatter(rows2d.at[slot], [iota + 112], diag_pad,
                               mask=pad_mask)
            plsc.store_scatter(vals2d.at[slot], [iota + 112],
                               jnp.zeros((16,), jnp.float32), mask=pad_mask)
            s2_flush(slot)

            @pl.when(gout >= 64)
            def _drain():
                s2_drainslot(lax.rem(gout - 64, 128))
        full = lo >= 112
        return (jnp.where(full, 0, lo), jnp.where(full, gout + 1, gout))

    lo, gout = lax.fori_loop(0, nedge // 16, s2_body, (0, 0))

    slot = lax.rem(gout, 128)
    for q in range(8):
        pm = (iota + q * 16) >= lo
        plsc.store_scatter(rows2d.at[slot], [iota + q * 16], diag_pad,
                           mask=pm)
        plsc.store_scatter(vals2d.at[slot], [iota + q * 16],
                           jnp.zeros((16,), jnp.float32), mask=pm)
    s2_flush(slot)
    gout = gout + 1

    outstanding = jnp.minimum(gout, 65)

    def s2_final_drain(i, _):
        @pl.when(i < outstanding)
        def _d():
            s2_drainslot(lax.rem(gout - 1 - i, 128))
        return _
    lax.fori_loop(0, 65, s2_final_drain, None)


def _sc_adjacency(idx_flat, n):
    mesh = plsc.VectorSubcoreMesh(
        core_axis_name="c", subcore_axis_name="s", num_cores=2,
        num_subcores=16)
    kern = functools.partial(
        pl.kernel,
        out_type=jax.ShapeDtypeStruct((n * n,), jnp.float32),
        mesh=mesh,
        compiler_params=pltpu.CompilerParams(needs_layout_passes=False),
        scratch_types=[
            pltpu.VMEM((n * _K,), jnp.int32),
            pltpu.VMEM((_GL, _GL), jnp.int32),
            pltpu.VMEM((_GL, _GL), jnp.float32),
            pltpu.VMEM((n // _NW,), jnp.float32),
            pltpu.VMEM((n // _NW,), jnp.float32),
            pltpu.VMEM((min(16384, n * n // _NW),), jnp.float32),
            pltpu.SemaphoreType.DMA,
            pltpu.SemaphoreType.DMA,
        ],
    )(functools.partial(_sc_adj_body, n=n))
    return kern(idx_flat)


@jax.jit
def kernel(embeddings):
    n, d = embeddings.shape
    rn = min(1024, n)
    xn = pl.pallas_call(
        _normalize_body,
        grid=(n // rn,),
        in_specs=[pl.BlockSpec((rn, d), lambda i: (i, 0))],
        out_specs=pl.BlockSpec((rn, d), lambda i: (i, 0)),
        out_shape=jax.ShapeDtypeStruct((n, d), jnp.float32),
    )(embeddings)

    strip = min(256, n)
    nstrips = n // strip
    idx = pl.pallas_call(
        functools.partial(_simtopk_body, strip=strip, n=n),
        grid=(nstrips,),
        in_specs=[
            pl.BlockSpec((strip, d), lambda i: (i, 0)),
            pl.BlockSpec((n, d), lambda i: (0, 0)),
        ],
        out_specs=pl.BlockSpec((strip, _K), lambda i: (i, 0)),
        out_shape=jax.ShapeDtypeStruct((n, _K), jnp.int32),
    )(xn, xn)

    idxt = idx.T
    adjacency = pl.pallas_call(
        functools.partial(_adjacency_body, strip=strip, n=n),
        grid=(nstrips,),
        in_specs=[
            pl.BlockSpec((strip, _K), lambda i: (i, 0)),
            pl.BlockSpec((_K, n), lambda i: (0, 0)),
        ],
        out_specs=pl.BlockSpec((strip, n), lambda i: (i, 0)),
        out_shape=jax.ShapeDtypeStruct((n, n), jnp.float32),
    )(idx, idxt)
    return adjacency, xn


# speedup vs baseline: 3.7530x; 1.0258x over previous
import functools

import jax
import jax.numpy as jnp
from jax import lax
from jax.experimental import pallas as pl
from jax.experimental.pallas import tpu as pltpu
from jax.experimental.pallas import tpu_sc as plsc

_K = 8
_NEG_INF = float("-inf")
_NW = 32
_SEGS = 4
_GL = 128


def _normalize_body(x_ref, o_ref):
    x = x_ref[...]
    n = jnp.sqrt(jnp.sum(x * x, axis=1, keepdims=True))
    o_ref[...] = x / jnp.maximum(n, 1e-12)


_SORT8 = [
    (0, 1), (2, 3), (4, 5), (6, 7),
    (0, 2), (1, 3), (4, 6), (5, 7),
    (1, 2), (5, 6), (0, 4), (1, 5),
    (2, 6), (3, 7), (2, 4), (3, 5),
    (1, 2), (3, 4), (5, 6),
]


def _ce(vals, chs, a, b):
    va, vb = vals[a], vals[b]
    g = va < vb
    vals[a], vals[b] = jnp.maximum(va, vb), jnp.minimum(va, vb)
    ca, cb = chs[a], chs[b]
    chs[a], chs[b] = jnp.where(g, cb, ca), jnp.where(g, ca, cb)


def _simtopk_body(a_ref, b_ref, idx_ref, *, strip: int, n: int):
    a = a_ref[...]
    b = b_ref[...]
    sim = jax.lax.dot_general(
        a, b, (((1,), (1,)), ((), ())), preferred_element_type=jnp.float32)
    i = pl.program_id(0)
    row_ids = jax.lax.broadcasted_iota(jnp.int32, (strip, n), 0) + i * strip
    col_ids = jax.lax.broadcasted_iota(jnp.int32, (strip, n), 1)
    sim = jnp.where(col_ids == row_ids, _NEG_INF, sim)

    ngrp = (n // 128) // _K
    tops = []
    for g in range(ngrp):
        vals = [sim[:, (g * _K + s) * 128:(g * _K + s + 1) * 128]
                for s in range(_K)]
        chs = [jnp.full((strip, 128), g * _K + s, jnp.int32)
               for s in range(_K)]
        for (x, y) in _SORT8:
            _ce(vals, chs, x, y)
        tops.append((vals, chs))
    while len(tops) > 1:
        nxt = []
        for p in range(0, len(tops), 2):
            (va, ca), (vb, cb) = tops[p], tops[p + 1]
            mv, mc = [], []
            for s in range(_K):
                x, y = va[s], vb[_K - 1 - s]
                gt = x < y
                mv.append(jnp.maximum(x, y))
                mc.append(jnp.where(gt, cb[_K - 1 - s], ca[s]))
            for stage in (4, 2, 1):
                for lo in range(_K):
                    if lo & stage == 0 and lo + stage < _K:
                        _ce(mv, mc, lo, lo + stage)
            nxt.append((mv, mc))
        tops = nxt
    vals, chs = tops[0]
    lane = jax.lax.broadcasted_iota(jnp.int32, (strip, 128), 1)
    cols = [chs[s] * 128 + lane for s in range(_K)]

    out_cols = []
    for _ in range(_K):
        m = vals[0]
        for s in range(1, _K):
            m = jnp.maximum(m, vals[s])
        m = jnp.max(m, axis=1, keepdims=True)
        amin = jnp.full((strip, 128), n, jnp.int32)
        for s in range(_K):
            amin = jnp.minimum(amin, jnp.where(vals[s] == m, cols[s], n))
        amax = jnp.min(amin, axis=1, keepdims=True)
        out_cols.append(amax)
        for s in range(_K):
            vals[s] = jnp.where(cols[s] == amax, _NEG_INF, vals[s])
    idx_ref[...] = jnp.concatenate(out_cols, axis=1)


def _adjacency_body(idx_strip_ref, idxt_ref, out_ref, *, strip: int, n: int):
    idx_strip = idx_strip_ref[...]
    idxt = idxt_ref[...]
    i0 = pl.program_id(0) * strip
    row_ids = jax.lax.broadcasted_iota(jnp.int32, (strip, n), 0) + i0
    col_ids = jax.lax.broadcasted_iota(jnp.int32, (strip, n), 1)
    acc = col_ids ^ idx_strip[:, 0:1]
    acc = jnp.minimum(acc, row_ids ^ idxt[0:1, :])
    for k in range(1, _K):
        acc = jnp.minimum(acc, col_ids ^ idx_strip[:, k:k + 1])
        acc = jnp.minimum(acc, row_ids ^ idxt[k:k + 1, :])
    a = jnp.where(acc == 0, 1.0, 0.0).astype(jnp.float32)
    rs = jnp.maximum(jnp.sum(a, axis=1, keepdims=True), 1e-8)
    out_ref[...] = a / rs


def _sc_adj_body(idx_hbm, out_hbm, idx_v, rows2d, vals2d, cnt_v, val_v,
                 zsrc, sem_fill, sem_sc, *, n: int):
    rpw = n // _NW
    nedge = n * _K
    seg_elems = nedge // _SEGS
    gps = seg_elems // _GL
    slab = rpw * n
    zlen = min(16384, slab)
    nfill = slab // zlen
    fwd_groups = (rpw * _K) // _GL

    c = lax.axis_index("c")
    s = lax.axis_index("s")
    w = s * 2 + c
    base = w * rpw
    iota = lax.iota(jnp.int32, 16)

    def zero_body(q, _):
        zsrc[pl.ds(q * 16, 16)] = jnp.zeros((16,), jnp.float32)
        return _
    lax.fori_loop(0, zlen // 16, zero_body, None)

    def fill_issue(i, _):
        pltpu.async_copy(zsrc, out_hbm.at[pl.ds(base * n + i * zlen, zlen)],
                         sem_fill)
        return _
    lax.fori_loop(0, nfill, fill_issue, None)

    pltpu.sync_copy(idx_hbm, idx_v)

    def cnt_init(q, _):
        cnt_v[pl.ds(q * 16, 16)] = jnp.full((16,), float(_K), jnp.float32)
        return _
    lax.fori_loop(0, rpw // 16, cnt_init, None)

    def s1_body(e16, _):
        e0 = e16 * 16
        t = idx_v[pl.ds(e0, 16)]
        inr = (t >= base) & (t < base + rpw)
        nhit = jnp.sum(jnp.where(inr, 1, 0))

        @pl.when(nhit > 0)
        def _do():
            src = lax.shift_right_logical(e0 + iota, 3)
            t8 = t * _K
            mut = plsc.load_gather(idx_v, [t8]) == src
            for k in range(1, _K):
                mut = mut | (plsc.load_gather(idx_v, [t8 + k]) == src)
            novel = jnp.where(inr & (~mut), 1.0, 0.0).astype(jnp.float32)
            tl = jnp.where(inr, t - base, 0)
            for l in range(16):
                plsc.addupdate_scatter(cnt_v, [tl], novel,
                                       mask=(iota == l) & inr)
        return _
    lax.fori_loop(0, nedge // 16, s1_body, None)

    def val_body(q, _):
        val_v[pl.ds(q * 16, 16)] = 1.0 / cnt_v[pl.ds(q * 16, 16)]
        return _
    lax.fori_loop(0, rpw // 16, val_body, None)

    def fill_drain(i, _):
        pltpu.make_async_copy(
            zsrc, out_hbm.at[pl.ds(base * n + i * zlen, zlen)],
            sem_fill).wait()
        return _
    lax.fori_loop(0, nfill, fill_drain, None)

    def fw_body(g, _):
        for q in range(_GL // 16):
            el = q * 16 + g * _GL
            t = idx_v[pl.ds(base * _K + el, 16)]
            i_loc = lax.shift_right_logical(el + iota, 3)
            rows2d[g, pl.ds(q * 16, 16)] = (base + i_loc) * n + t
            vals2d[g, pl.ds(q * 16, 16)] = plsc.load_gather(val_v, [i_loc])
        return _
    lax.fori_loop(0, fwd_groups, fw_body, None)

    def fw_issue(g, _):
        pltpu.async_copy(vals2d.at[g], out_hbm.at[rows2d.at[g]], sem_sc)
        return _
    lax.fori_loop(0, fwd_groups, fw_issue, None)

    def fw_drain(g, _):
        pltpu.make_async_copy(vals2d.at[g], out_hbm.at[rows2d.at[g]],
                              sem_sc).wait()
        return _
    lax.fori_loop(0, fwd_groups, fw_drain, None)

    diag_pad = (base + iota) * (n + 1)

    def s2_flush(slot):
        pltpu.async_copy(vals2d.at[slot], out_hbm.at[rows2d.at[slot]],
                         sem_sc)

    def s2_drainslot(slot):
        pltpu.make_async_copy(vals2d.at[slot], out_hbm.at[rows2d.at[slot]],
                              sem_sc).wait()

    def s2_body(e16, carry):
        lo, gout = carry
        e0 = e16 * 16
        t = idx_v[pl.ds(e0, 16)]
        src = lax.shift_right_logical(e0 + iota, 3)
        inr = (t >= base) & (t < base + rpw)
        nhit = jnp.sum(jnp.where(inr, 1, 0))
        slot = lax.rem(gout, 128)
        tl = jnp.where(inr, t - base, 0)
        off = t * n + src
        v = plsc.load_gather(val_v, [tl])
        plsc.store_compressed(rows2d.at[slot, pl.ds(lo, 16)], off, mask=inr)
        plsc.store_compressed(vals2d.at[slot, pl.ds(lo, 16)], v, mask=inr)
        lo = lo + nhit

        @pl.when(lo >= 112)
        def _flush():
            pad_mask = (iota + 112) >= lo
            plsc.store_scatter(rows2d.at[slot], [iota + 112], diag_pad,
                               mask=pad_mask)
            plsc.store_scatter(vals2d.at[slot], [iota + 112],
                               jnp.zeros((16,), jnp.float32), mask=pad_mask)
            s2_flush(slot)

            @pl.when(gout >= 64)
            def _drain():
                s2_drainslot(lax.rem(gout - 64, 128))
        full = lo >= 112
        return (jnp.where(full, 0, lo), jnp.where(full, gout + 1, gout))

    lo, gout = lax.fori_loop(0, nedge // 16, s2_body, (0, 0))

    slot = lax.rem(gout, 128)
    for q in range(8):
        pm = (iota + q * 16) >= lo
        plsc.store_scatter(rows2d.at[slot], [iota + q * 16], diag_pad,
                           mask=pm)
        plsc.store_scatter(vals2d.at[slot], [iota + q * 16],
                           jnp.zeros((16,), jnp.float32), mask=pm)
    s2_flush(slot)
    gout = gout + 1

    outstanding = jnp.minimum(gout, 65)

    def s2_final_drain(i, _):
        @pl.when(i < outstanding)
        def _d():
            s2_drainslot(lax.rem(gout - 1 - i, 128))
        return _
    lax.fori_loop(0, 65, s2_final_drain, None)


def _sc_adjacency(idx_flat, n):
    mesh = plsc.VectorSubcoreMesh(
        core_axis_name="c", subcore_axis_name="s", num_cores=2,
        num_subcores=16)
    kern = functools.partial(
        pl.kernel,
        out_type=jax.ShapeDtypeStruct((n * n,), jnp.float32),
        mesh=mesh,
        compiler_params=pltpu.CompilerParams(needs_layout_passes=False),
        scratch_types=[
            pltpu.VMEM((n * _K,), jnp.int32),
            pltpu.VMEM((_GL, _GL), jnp.int32),
            pltpu.VMEM((_GL, _GL), jnp.float32),
            pltpu.VMEM((n // _NW,), jnp.float32),
            pltpu.VMEM((n // _NW,), jnp.float32),
            pltpu.VMEM((min(16384, n * n // _NW),), jnp.float32),
            pltpu.SemaphoreType.DMA,
            pltpu.SemaphoreType.DMA,
        ],
    )(functools.partial(_sc_adj_body, n=n))
    return kern(idx_flat)


@jax.jit
def kernel(embeddings):
    n, d = embeddings.shape
    rn = min(1024, n)
    xn = pl.pallas_call(
        _normalize_body,
        grid=(n // rn,),
        in_specs=[pl.BlockSpec((rn, d), lambda i: (i, 0))],
        out_specs=pl.BlockSpec((rn, d), lambda i: (i, 0)),
        out_shape=jax.ShapeDtypeStruct((n, d), jnp.float32),
    )(embeddings)

    strip = min(256, n)
    nstrips = n // strip
    idx = pl.pallas_call(
        functools.partial(_simtopk_body, strip=strip, n=n),
        grid=(nstrips,),
        in_specs=[
            pl.BlockSpec((strip, d), lambda i: (i, 0)),
            pl.BlockSpec((n, d), lambda i: (0, 0)),
        ],
        out_specs=pl.BlockSpec((strip, _K), lambda i: (i, 0)),
        out_shape=jax.ShapeDtypeStruct((n, _K), jnp.int32),
    )(xn, xn)

    idxt = idx.T
    adjacency = pl.pallas_call(
        functools.partial(_adjacency_body, strip=strip, n=n),
        grid=(nstrips,),
        in_specs=[
            pl.BlockSpec((strip, _K), lambda i: (i, 0)),
            pl.BlockSpec((_K, n), lambda i: (0, 0)),
        ],
        out_specs=pl.BlockSpec((strip, n), lambda i: (i, 0)),
        out_shape=jax.ShapeDtypeStruct((n, n), jnp.float32),
    )(idx, idxt)
    return adjacency, xn
